# baseline (speedup 1.0000x reference)
import functools

import numpy as np
import jax
import jax.numpy as jnp
from jax import lax
from jax.experimental import pallas as pl
from jax.experimental.pallas import tpu as pltpu
from jax.experimental.pallas import tpu_sc as plsc

SEQ = 4096
DIM = 128
_LANES = 16
_NUM_CORES = 2
_NUM_SUBCORES = 16
_NW = _NUM_CORES * _NUM_SUBCORES
_B_PER_W = SEQ // _NW
_NGROUP = DIM // 32
_WPR = DIM // 2


def _pe_table() -> np.ndarray:
    pos = np.arange(1, SEQ + 1, dtype=np.float64)[:, None]
    i = np.arange(1, DIM + 1, dtype=np.float64)[None, :]
    w_even = (1.0 / 10000.0) ** (i / DIM)
    w_odd = (1.0 / 10000.0) ** ((i - 1.0) / DIM)
    even = (np.arange(1, DIM + 1) % 2 == 0)[None, :]
    return np.where(even, np.sin(pos * w_even), np.cos(pos * w_odd)).astype(
        np.float32
    )


def _pe_packed() -> np.ndarray:
    u = _pe_table().view(np.uint32)
    rounded = (u + 0x7FFF + ((u >> 16) & 1)) >> 16
    g = rounded.reshape(SEQ, _NGROUP, 2, _LANES).astype(np.uint32)
    words = g[:, :, 0, :] | (g[:, :, 1, :] << 16)
    return words.reshape(SEQ * _WPR).view(np.int32)


_PE_NP = _pe_packed()

_mesh = plsc.VectorSubcoreMesh(core_axis_name="c", subcore_axis_name="s")


@functools.partial(
    pl.kernel,
    mesh=_mesh,
    out_type=jax.ShapeDtypeStruct((SEQ, DIM), jnp.float32),
    scratch_types=[
        pltpu.VMEM((_B_PER_W,), jnp.int32),
        pltpu.VMEM((_B_PER_W, DIM), jnp.float32),
        pltpu.VMEM((_B_PER_W * _WPR,), jnp.int32),
        pltpu.SemaphoreType.DMA,
        pltpu.SemaphoreType.DMA,
    ],
)
def _emb_pe_kernel(x_hbm, table_hbm, pe_hbm, out_hbm, idx_v, rows_v, pe_v,
                   sem_g, sem_p):
    wid = lax.axis_index("s") * _NUM_CORES + lax.axis_index("c")
    base = wid * _B_PER_W

    pltpu.sync_copy(x_hbm.at[pl.ds(base, _B_PER_W)], idx_v)
    gather = pltpu.async_copy(table_hbm.at[idx_v], rows_v, sem_g)
    pe_cp = pltpu.async_copy(
        pe_hbm.at[pl.ds(base * _WPR, _B_PER_W * _WPR)], pe_v, sem_p
    )
    gather.wait()
    pe_cp.wait()

    hi_mask = jnp.full((_LANES,), -65536, dtype=jnp.int32)

    def add_row(i, _):
        woff = i * _WPR
        for j in range(_NGROUP):
            w = pe_v[pl.ds(woff + j * _LANES, _LANES)]
            lo = lax.bitcast_convert_type(lax.shift_left(w, 16), jnp.float32)
            hi = lax.bitcast_convert_type(
                lax.bitwise_and(w, hi_mask), jnp.float32
            )
            plsc.addupdate(rows_v.at[i, pl.ds(j * 32, _LANES)], lo)
            plsc.addupdate(rows_v.at[i, pl.ds(j * 32 + _LANES, _LANES)], hi)
        return ()

    lax.fori_loop(0, _B_PER_W, add_row, ())

    pltpu.sync_copy(rows_v, out_hbm.at[pl.ds(base, _B_PER_W)])


def kernel(x, table):
    pe = jnp.asarray(_PE_NP)
    return _emb_pe_kernel(x.astype(jnp.int32), table, pe)


# speedup vs baseline: 1.0318x; 1.0318x over previous
import functools

import numpy as np
import jax
import jax.numpy as jnp
from jax import lax
from jax.experimental import pallas as pl
from jax.experimental.pallas import tpu as pltpu
from jax.experimental.pallas import tpu_sc as plsc

SEQ = 4096
DIM = 128
_LANES = 16
_NUM_CORES = 2
_NUM_SUBCORES = 16
_NW = _NUM_CORES * _NUM_SUBCORES
_B_PER_W = SEQ // _NW


def _pe_table() -> np.ndarray:
    pos = np.arange(1, SEQ + 1, dtype=np.float64)[:, None]
    i = np.arange(1, DIM + 1, dtype=np.float64)[None, :]
    w_even = (1.0 / 10000.0) ** (i / DIM)
    w_odd = (1.0 / 10000.0) ** ((i - 1.0) / DIM)
    even = (np.arange(1, DIM + 1) % 2 == 0)[None, :]
    return np.where(even, np.sin(pos * w_even), np.cos(pos * w_odd)).astype(
        np.float32
    )


_PE_NP = _pe_table().reshape(SEQ * DIM)

_mesh = plsc.VectorSubcoreMesh(core_axis_name="c", subcore_axis_name="s")


@functools.partial(
    pl.kernel,
    mesh=_mesh,
    out_type=jax.ShapeDtypeStruct((SEQ, DIM), jnp.float32),
    scratch_types=[
        pltpu.VMEM((_B_PER_W,), jnp.int32),
        pltpu.VMEM((_B_PER_W, DIM), jnp.float32),
        pltpu.VMEM((_B_PER_W * DIM,), jnp.float32),
        pltpu.SemaphoreType.DMA,
        pltpu.SemaphoreType.DMA,
    ],
)
def _emb_pe_kernel(x_hbm, table_hbm, pe_hbm, out_hbm, idx_v, rows_v, pe_v,
                   sem_g, sem_p):
    wid = lax.axis_index("s") * _NUM_CORES + lax.axis_index("c")
    base = wid * _B_PER_W

    pltpu.sync_copy(x_hbm.at[pl.ds(base, _B_PER_W)], idx_v)
    gather = pltpu.async_copy(table_hbm.at[idx_v], rows_v, sem_g)
    pe_cp = pltpu.async_copy(
        pe_hbm.at[pl.ds(base * DIM, _B_PER_W * DIM)], pe_v, sem_p
    )
    gather.wait()
    pe_cp.wait()

    @plsc.parallel_loop(0, _B_PER_W, 1, unroll=2)
    def _(i):
        for j in range(DIM // _LANES):
            plsc.addupdate(
                rows_v.at[i, pl.ds(j * _LANES, _LANES)],
                pe_v[pl.ds(i * DIM + j * _LANES, _LANES)],
            )

    pltpu.sync_copy(rows_v, out_hbm.at[pl.ds(base, _B_PER_W)])


def kernel(x, table):
    pe = jnp.asarray(_PE_NP)
    return _emb_pe_kernel(x.astype(jnp.int32), table, pe)


# speedup vs baseline: 1.0689x; 1.0360x over previous
import functools

import numpy as np
import jax
import jax.numpy as jnp
from jax import lax
from jax.experimental import pallas as pl
from jax.experimental.pallas import tpu as pltpu
from jax.experimental.pallas import tpu_sc as plsc

SEQ = 4096
DIM = 128
_LANES = 16
_NUM_CORES = 2
_NUM_SUBCORES = 16
_NW = _NUM_CORES * _NUM_SUBCORES
_B_PER_W = SEQ // _NW
_NGROUP = DIM // 32
_WPR = DIM // 2


def _pe_table() -> np.ndarray:
    pos = np.arange(1, SEQ + 1, dtype=np.float64)[:, None]
    i = np.arange(1, DIM + 1, dtype=np.float64)[None, :]
    w_even = (1.0 / 10000.0) ** (i / DIM)
    w_odd = (1.0 / 10000.0) ** ((i - 1.0) / DIM)
    even = (np.arange(1, DIM + 1) % 2 == 0)[None, :]
    return np.where(even, np.sin(pos * w_even), np.cos(pos * w_odd)).astype(
        np.float32
    )


def _pe_packed() -> np.ndarray:
    u = _pe_table().view(np.uint32)
    rounded = (u + 0x7FFF + ((u >> 16) & 1)) >> 16
    g = rounded.reshape(SEQ, _NGROUP, 2, _LANES).astype(np.uint32)
    words = g[:, :, 0, :] | (g[:, :, 1, :] << 16)
    return words.reshape(SEQ * _WPR).view(np.int32)


_PE_NP = _pe_packed()

_mesh = plsc.VectorSubcoreMesh(core_axis_name="c", subcore_axis_name="s")


@functools.partial(
    pl.kernel,
    mesh=_mesh,
    out_type=jax.ShapeDtypeStruct((SEQ, DIM), jnp.float32),
    scratch_types=[
        pltpu.VMEM((_B_PER_W,), jnp.int32),
        pltpu.VMEM((_B_PER_W, DIM), jnp.float32),
        pltpu.VMEM((_B_PER_W * _WPR,), jnp.int32),
        pltpu.SemaphoreType.DMA,
        pltpu.SemaphoreType.DMA,
    ],
)
def _emb_pe_kernel(x_hbm, table_hbm, pe_hbm, out_hbm, idx_v, rows_v, pe_v,
                   sem_g, sem_p):
    wid = lax.axis_index("s") * _NUM_CORES + lax.axis_index("c")
    base = wid * _B_PER_W

    pltpu.sync_copy(x_hbm.at[pl.ds(base, _B_PER_W)], idx_v)
    gather = pltpu.async_copy(table_hbm.at[idx_v], rows_v, sem_g)
    pe_cp = pltpu.async_copy(
        pe_hbm.at[pl.ds(base * _WPR, _B_PER_W * _WPR)], pe_v, sem_p
    )
    gather.wait()
    pe_cp.wait()

    hi_mask = jnp.full((_LANES,), -65536, dtype=jnp.int32)

    @plsc.parallel_loop(0, _B_PER_W, 1, unroll=2)
    def _(i):
        woff = i * _WPR
        for j in range(_NGROUP):
            w = pe_v[pl.ds(woff + j * _LANES, _LANES)]
            lo = lax.bitcast_convert_type(lax.shift_left(w, 16), jnp.float32)
            hi = lax.bitcast_convert_type(
                lax.bitwise_and(w, hi_mask), jnp.float32
            )
            plsc.addupdate(rows_v.at[i, pl.ds(j * 32, _LANES)], lo)
            plsc.addupdate(rows_v.at[i, pl.ds(j * 32 + _LANES, _LANES)], hi)

    pltpu.sync_copy(rows_v, out_hbm.at[pl.ds(base, _B_PER_W)])


def kernel(x, table):
    pe = jnp.asarray(_PE_NP)
    return _emb_pe_kernel(x.astype(jnp.int32), table, pe)


# speedup vs baseline: 1.0813x; 1.0115x over previous
import functools

import numpy as np
import jax
import jax.numpy as jnp
from jax import lax
from jax.experimental import pallas as pl
from jax.experimental.pallas import tpu as pltpu
from jax.experimental.pallas import tpu_sc as plsc

SEQ = 4096
DIM = 128
_LANES = 16
_NUM_CORES = 2
_NUM_SUBCORES = 16
_NW = _NUM_CORES * _NUM_SUBCORES
_B_PER_W = SEQ // _NW
_NGROUP = DIM // 32
_WPR = DIM // 2


def _pe_table() -> np.ndarray:
    pos = np.arange(1, SEQ + 1, dtype=np.float64)[:, None]
    i = np.arange(1, DIM + 1, dtype=np.float64)[None, :]
    w_even = (1.0 / 10000.0) ** (i / DIM)
    w_odd = (1.0 / 10000.0) ** ((i - 1.0) / DIM)
    even = (np.arange(1, DIM + 1) % 2 == 0)[None, :]
    return np.where(even, np.sin(pos * w_even), np.cos(pos * w_odd)).astype(
        np.float32
    )


def _pe_packed() -> np.ndarray:
    u = _pe_table().view(np.uint32)
    rounded = (u + 0x7FFF + ((u >> 16) & 1)) >> 16
    g = rounded.reshape(SEQ, _NGROUP, 2, _LANES).astype(np.uint32)
    words = g[:, :, 0, :] | (g[:, :, 1, :] << 16)
    return words.reshape(SEQ * _WPR).view(np.int32)


_PE_NP = _pe_packed()

_mesh = plsc.VectorSubcoreMesh(core_axis_name="c", subcore_axis_name="s")


@functools.partial(
    pl.kernel,
    mesh=_mesh,
    out_type=jax.ShapeDtypeStruct((SEQ, DIM), jnp.float32),
    scratch_types=[
        pltpu.VMEM((_B_PER_W,), jnp.int32),
        pltpu.VMEM((_B_PER_W, DIM), jnp.float32),
        pltpu.VMEM((_B_PER_W * _WPR,), jnp.int32),
        pltpu.SemaphoreType.DMA,
        pltpu.SemaphoreType.DMA,
    ],
)
def _emb_pe_kernel(x_hbm, table_hbm, pe_hbm, out_hbm, idx_v, rows_v, pe_v,
                   sem_g, sem_p):
    wid = lax.axis_index("s") * _NUM_CORES + lax.axis_index("c")
    base = wid * _B_PER_W

    pe_cp = pltpu.async_copy(
        pe_hbm.at[pl.ds(base * _WPR, _B_PER_W * _WPR)], pe_v, sem_p
    )
    pltpu.sync_copy(x_hbm.at[pl.ds(base, _B_PER_W)], idx_v)
    gather = pltpu.async_copy(table_hbm.at[idx_v], rows_v, sem_g)
    gather.wait()
    pe_cp.wait()

    hi_mask = jnp.full((_LANES,), -65536, dtype=jnp.int32)

    @plsc.parallel_loop(0, _B_PER_W, 1, unroll=2)
    def _(i):
        woff = i * _WPR
        for j in range(_NGROUP):
            w = pe_v[pl.ds(woff + j * _LANES, _LANES)]
            lo = lax.bitcast_convert_type(lax.shift_left(w, 16), jnp.float32)
            hi = lax.bitcast_convert_type(
                lax.bitwise_and(w, hi_mask), jnp.float32
            )
            plsc.addupdate(rows_v.at[i, pl.ds(j * 32, _LANES)], lo)
            plsc.addupdate(rows_v.at[i, pl.ds(j * 32 + _LANES, _LANES)], hi)

    pltpu.sync_copy(rows_v, out_hbm.at[pl.ds(base, _B_PER_W)])


def kernel(x, table):
    pe = jnp.asarray(_PE_NP)
    return _emb_pe_kernel(x.astype(jnp.int32), table, pe)
